# Initial kernel scaffold; baseline (speedup 1.0000x reference)
import jax
import jax.numpy as jnp
from jax.experimental import pallas as pl


def kernel(token_indices, embedding_table):
    raise NotImplementedError("write your pallas kernel here")


# speedup vs baseline: 1.4589x; 1.4589x over previous
import functools

import jax
import jax.numpy as jnp
from jax import lax
from jax.experimental import pallas as pl
from jax.experimental.pallas import tpu as pltpu
from jax.experimental.pallas import tpu_sc as plsc

_INFO = plsc.get_sparse_core_info()
_NC = _INFO.num_cores
_NS = _INFO.num_subcores
_NW = _NC * _NS


@functools.partial(jax.jit, static_argnums=(2, 3))
def _embedding_lookup(table, idx, b_per_w, chunk):
    n_chunks = b_per_w // chunk
    B = idx.shape[0]
    D = table.shape[1]
    mesh = plsc.VectorSubcoreMesh(core_axis_name="c", subcore_axis_name="s")

    @functools.partial(
        pl.kernel,
        out_type=jax.ShapeDtypeStruct((B, D), jnp.float32),
        mesh=mesh,
        compiler_params=pltpu.CompilerParams(use_tc_tiling_on_sc=False),
        scratch_types=[
            pltpu.VMEM((chunk,), jnp.int32),
            pltpu.VMEM((chunk, D), jnp.float32),
            pltpu.SemaphoreType.DMA,
        ],
    )
    def emb(table_hbm, idx_hbm, out_hbm, idx_v, rows_v, sem):
        wid = lax.axis_index("s") * _NC + lax.axis_index("c")
        base = wid * b_per_w

        @pl.loop(0, n_chunks)
        def body(i):
            off = base + i * chunk
            pltpu.sync_copy(idx_hbm.at[pl.ds(off, chunk)], idx_v)
            pltpu.async_copy(table_hbm.at[idx_v], rows_v, sem).wait()
            pltpu.sync_copy(rows_v, out_hbm.at[pl.ds(off, chunk)])

    return emb(table, idx)


def kernel(token_indices, embedding_table):
    Bt, S = token_indices.shape
    V, D = embedding_table.shape
    B = Bt * S
    idx = token_indices.reshape(B).astype(jnp.int32)
    b_per_w = B // _NW
    out = _embedding_lookup(embedding_table, idx, b_per_w, 1024)
    return out.reshape(Bt, S, D)


# speedup vs baseline: 1.5008x; 1.0287x over previous
import functools

import jax
import jax.numpy as jnp
from jax import lax
from jax.experimental import pallas as pl
from jax.experimental.pallas import tpu as pltpu
from jax.experimental.pallas import tpu_sc as plsc

_INFO = plsc.get_sparse_core_info()
_NC = _INFO.num_cores
_NS = _INFO.num_subcores
_NW = _NC * _NS


@functools.partial(jax.jit, static_argnums=(2, 3, 4))
def _embedding_lookup(table, idx, n_chunks, chunk, nbuf):
    B = idx.shape[0] * idx.shape[1] * idx.shape[2]
    D = table.shape[1]
    b_per_w = n_chunks * chunk
    mesh = plsc.VectorSubcoreMesh(core_axis_name="c", subcore_axis_name="s")

    @functools.partial(
        pl.kernel,
        out_type=jax.ShapeDtypeStruct((B, D), jnp.float32),
        mesh=mesh,
        compiler_params=pltpu.CompilerParams(use_tc_tiling_on_sc=False),
        scratch_types=[
            pltpu.VMEM((n_chunks, chunk), jnp.int32),
            pltpu.VMEM((nbuf, chunk, D), jnp.float32),
            pltpu.SemaphoreType.DMA((nbuf,)),
            pltpu.SemaphoreType.DMA((nbuf,)),
        ],
    )
    def emb(table_hbm, idx_hbm, out_hbm, idx_v, rows_v, gsem, wsem):
        wid = lax.axis_index("s") * _NC + lax.axis_index("c")
        base = wid * b_per_w

        def start_gather(i, b):
            pltpu.async_copy(table_hbm.at[idx_v.at[i]], rows_v.at[b],
                             gsem.at[b])

        def wait_gather(i, b):
            pltpu.make_async_copy(table_hbm.at[idx_v.at[i]], rows_v.at[b],
                                  gsem.at[b]).wait()

        def start_wb(i, b):
            pltpu.async_copy(rows_v.at[b],
                             out_hbm.at[pl.ds(base + i * chunk, chunk)],
                             wsem.at[b])

        def wait_wb(i, b):
            pltpu.make_async_copy(rows_v.at[b],
                                  out_hbm.at[pl.ds(base + i * chunk, chunk)],
                                  wsem.at[b]).wait()

        pltpu.sync_copy(idx_hbm.at[wid], idx_v)

        for b in range(nbuf):
            start_gather(b, b)

        @pl.loop(0, n_chunks - nbuf, step=nbuf)
        def ring(g):
            for b in range(nbuf):
                i = g + b
                wait_gather(i, b)
                start_wb(i, b)
                wait_wb(i, b)
                start_gather(i + nbuf, b)

        for b in range(nbuf):
            i = n_chunks - nbuf + b
            wait_gather(i, b)
            start_wb(i, b)
        for b in range(nbuf):
            wait_wb(n_chunks - nbuf + b, b)

    return emb(table, idx)


def kernel(token_indices, embedding_table):
    Bt, S = token_indices.shape
    V, D = embedding_table.shape
    B = Bt * S
    chunk, nbuf = 1280, 2
    n_chunks = B // (_NW * chunk)
    idx = token_indices.reshape(_NW, n_chunks, chunk).astype(jnp.int32)
    out = _embedding_lookup(embedding_table, idx, n_chunks, chunk, nbuf)
    return out.reshape(Bt, S, D)
